# baseline (speedup 1.0000x reference)
import functools

import jax
import jax.numpy as jnp
from jax import lax
from jax.experimental import pallas as pl
from jax.experimental.pallas import tpu as pltpu


def _ring_body(m_hbm, x_hbm, o_hbm, mbuf, xbuf, msem, xsem, osem,
               *, nbuf, ch, nch):
    j = pl.program_id(0)

    def _slot(c):
        return c % nbuf if isinstance(c, int) else lax.rem(c, nbuf)

    def in_cp(c):
        return pltpu.make_async_copy(
            x_hbm.at[pl.ds(c * ch, ch)], xbuf.at[_slot(c)], xsem.at[_slot(c)])

    def out_cp(c):
        return pltpu.make_async_copy(
            xbuf.at[_slot(c)], o_hbm.at[pl.ds(c * ch, ch)], osem.at[_slot(c)])

    @pl.when(j == 0)
    def _():
        pltpu.make_async_copy(m_hbm, mbuf, msem).start()
        for c in range(nbuf):
            in_cp(c).start()
        pltpu.make_async_copy(m_hbm, mbuf, msem).wait()

    @pl.when((j > 0) & (j + nbuf - 1 < nch))
    def _():
        out_cp(j - 1).wait()
        in_cp(j + nbuf - 1).start()

    in_cp(j).wait()
    slot = _slot(j)
    xbuf[slot] = xbuf[slot] * mbuf[pl.ds(j * ch // 32, 1)]
    out_cp(j).start()

    @pl.when(j == nch - 1)
    def _():
        for c in range(nch - nbuf, nch):
            out_cp(c).wait()


def kernel(input, mask):
    B, C, K, H, W = input.shape
    BC = B * C
    x = input.reshape(BC * K, H, W)
    m = mask.reshape(BC, H, W)

    CH = 16
    NBUF = 14
    nch = (BC * K) // CH

    out = pl.pallas_call(
        functools.partial(_ring_body, nbuf=NBUF, ch=CH, nch=nch),
        grid=(nch,),
        in_specs=[
            pl.BlockSpec(memory_space=pl.ANY),
            pl.BlockSpec(memory_space=pl.ANY),
        ],
        out_specs=pl.BlockSpec(memory_space=pl.ANY),
        out_shape=jax.ShapeDtypeStruct((BC * K, H, W), x.dtype),
        scratch_shapes=[
            pltpu.VMEM((BC, H, W), x.dtype),
            pltpu.VMEM((NBUF, CH, H, W), x.dtype),
            pltpu.SemaphoreType.DMA,
            pltpu.SemaphoreType.DMA((NBUF,)),
            pltpu.SemaphoreType.DMA((NBUF,)),
        ],
        compiler_params=pltpu.CompilerParams(
            dimension_semantics=("arbitrary",),
            vmem_limit_bytes=63 * 1024 * 1024,
        ),
    )(m, x)
    return out.reshape(B, C, K, H, W)


# speedup vs baseline: 1.0640x; 1.0640x over previous
import jax
import jax.numpy as jnp
from jax.experimental import pallas as pl
from jax.experimental.pallas import tpu as pltpu


def _body(m_ref, x_ref, o_ref):
    g, h, w = x_ref.shape
    mg = m_ref.shape[0]
    x = x_ref[...].reshape(mg, g // mg, h, w)
    o_ref[...] = (x * m_ref[...][:, None]).reshape(g, h, w)


def kernel(input, mask):
    B, C, K, H, W = input.shape
    BC = B * C
    x = input.reshape(BC * K, H, W)
    m = mask.reshape(BC, H, W)

    ROWS = 64
    n = (BC * K) // ROWS

    out = pl.pallas_call(
        _body,
        grid=(n,),
        in_specs=[
            pl.BlockSpec((ROWS // K, H, W), lambda j: (j, 0, 0)),
            pl.BlockSpec((ROWS, H, W), lambda j: (j, 0, 0)),
        ],
        out_specs=pl.BlockSpec((ROWS, H, W), lambda j: (j, 0, 0)),
        out_shape=jax.ShapeDtypeStruct((BC * K, H, W), x.dtype),
        compiler_params=pltpu.CompilerParams(
            dimension_semantics=("arbitrary",),
            vmem_limit_bytes=110 * 1024 * 1024,
        ),
    )(m, x)
    return out.reshape(B, C, K, H, W)
